# baseline (speedup 1.0000x reference)
import functools

import jax
import jax.numpy as jnp
from jax import lax
from jax.experimental import pallas as pl
from jax.experimental.pallas import tpu as pltpu
from jax.experimental.pallas import tpu_sc as plsc

ENT_DIM = 128
REL_DIM = 384
EMBEDDING_RANGE = 0.109375
GAMMA = 12.0
PI = 3.141592653589793

NC = 2
NS = 16
NW = NC * NS
CHUNK = 64
NBUF = 2
L = 16


def _rel_prep_body(rel_ref, out_ref):
    rel = rel_ref[...]
    scale = rel[:, 0:ENT_DIM]
    translate = rel[:, ENT_DIM:2 * ENT_DIM]
    theta = rel[:, 2 * ENT_DIM:2 * ENT_DIM + ENT_DIM // 2]
    theta = theta * jnp.float32(PI / EMBEDDING_RANGE)
    r64 = lax.broadcasted_iota(jnp.int32, (ENT_DIM // 2, ENT_DIM), 0)
    c128 = lax.broadcasted_iota(jnp.int32, (ENT_DIM // 2, ENT_DIM), 1)
    p_dup = (c128 // 2 == r64).astype(jnp.float32)
    dot = functools.partial(
        jax.lax.dot_general,
        dimension_numbers=(((1,), (0,)), ((), ())),
        precision=jax.lax.Precision.HIGHEST,
    )
    cos_d = dot(jnp.cos(theta), p_dup)
    sin_d = dot(jnp.sin(theta), p_dup)
    lane = lax.broadcasted_iota(jnp.int32, (1, ENT_DIM), 1)
    sgn = jnp.where(lane % 2 == 0, jnp.float32(-1.0), jnp.float32(1.0))
    out_ref[...] = jnp.concatenate(
        [cos_d * scale, sgn * sin_d * scale, translate * scale], axis=1)


def _rel_prep(rel_table):
    n = rel_table.shape[0]
    return pl.pallas_call(
        _rel_prep_body,
        out_shape=jax.ShapeDtypeStruct((n, REL_DIM), jnp.float32),
    )(rel_table)


def _vec_rsqrt(s):
    one = jnp.full((L,), 1.0, jnp.float32)
    y = (one + one) / (one + s)
    half = jnp.full((L,), 0.5, jnp.float32) * s
    c15 = jnp.full((L,), 1.5, jnp.float32)
    for _ in range(7):
        y = y * (c15 - half * y * y)
    return y


def _lane_gather(vec, idx):
    return lax.gather(
        vec, idx[:, None],
        dimension_numbers=lax.GatherDimensionNumbers(
            offset_dims=(), collapsed_slice_dims=(0,), start_index_map=(0,)),
        slice_sizes=(1,),
        mode=lax.GatherScatterMode.PROMISE_IN_BOUNDS)


def _lane_sum(v, lane_iota):
    for sh in (8, 4, 2, 1):
        idx = lane_iota ^ jnp.full((L,), sh, jnp.int32)
        v = v + _lane_gather(v, idx)
    return v


def _sc_score(ent_table, rel_proc, h3, t3, r3, batch):
    b_per_w = batch // NW
    n_chunks = b_per_w // CHUNK
    mesh = plsc.VectorSubcoreMesh(core_axis_name="c", subcore_axis_name="s")

    @functools.partial(
        pl.kernel,
        mesh=mesh,
        out_type=jax.ShapeDtypeStruct((NW, b_per_w), jnp.float32),
        scratch_types=[
            pltpu.VMEM((n_chunks, CHUNK), jnp.int32),
            pltpu.VMEM((n_chunks, CHUNK), jnp.int32),
            pltpu.VMEM((n_chunks, CHUNK), jnp.int32),
            pltpu.VMEM((NBUF, CHUNK, ENT_DIM), jnp.float32),
            pltpu.VMEM((NBUF, CHUNK, ENT_DIM), jnp.float32),
            pltpu.VMEM((NBUF, CHUNK, REL_DIM), jnp.float32),
            pltpu.VMEM((b_per_w,), jnp.float32),
        ]
        + [pltpu.SemaphoreType.DMA] * (3 * NBUF),
    )
    def score_kernel(ent_hbm, rel_hbm, h_hbm, t_hbm, r_hbm, out_hbm,
                     hidx, tidx, ridx, hbuf, tbuf, rbuf, sbuf, *sems):
        wid = lax.axis_index("s") * NC + lax.axis_index("c")
        pltpu.sync_copy(h_hbm.at[wid], hidx)
        pltpu.sync_copy(t_hbm.at[wid], tidx)
        pltpu.sync_copy(r_hbm.at[wid], ridx)

        def start_gather(c, b):
            pltpu.make_async_copy(
                ent_hbm.at[hidx.at[c]], hbuf.at[b], sems[b]).start()
            pltpu.make_async_copy(
                ent_hbm.at[tidx.at[c]], tbuf.at[b], sems[NBUF + b]).start()
            pltpu.make_async_copy(
                rel_hbm.at[ridx.at[c]], rbuf.at[b], sems[2 * NBUF + b]).start()

        def wait_gather(b):
            pltpu.make_async_copy(
                ent_hbm.at[hidx.at[0]], hbuf.at[b], sems[b]).wait()
            pltpu.make_async_copy(
                ent_hbm.at[tidx.at[0]], tbuf.at[b], sems[NBUF + b]).wait()
            pltpu.make_async_copy(
                rel_hbm.at[ridx.at[0]], rbuf.at[b], sems[2 * NBUF + b]).wait()

        lane_iota = lax.iota(jnp.int32, L)
        one_i = jnp.full((L,), 1, jnp.int32)
        swap_idx = lane_iota ^ one_i
        eps = jnp.full((L,), 1e-12, jnp.float32)
        gamma = jnp.full((L,), GAMMA, jnp.float32)

        def row_score(hrow, trow, rrow):
            hv, tv = [], []
            hh = jnp.zeros((L,), jnp.float32)
            tt = jnp.zeros((L,), jnp.float32)
            for k in range(ENT_DIM // L):
                hk = hrow[pl.ds(k * L, L)]
                tk = trow[pl.ds(k * L, L)]
                hv.append(hk)
                tv.append(tk)
                hh = hh + hk * hk
                tt = tt + tk * tk
            sh = _lane_sum(hh, lane_iota)
            st = _lane_sum(tt, lane_iota)
            one = jnp.full((L,), 1.0, jnp.float32)
            inv_h = one / jnp.maximum(sh * _vec_rsqrt(sh), eps)
            inv_t = one / jnp.maximum(st * _vec_rsqrt(st), eps)
            acc = jnp.zeros((L,), jnp.float32)
            for k in range(ENT_DIM // L):
                tn = tv[k] * inv_t
                tsw = _lane_gather(tn, swap_idx)
                out = rrow[pl.ds(k * L, L)] * tn \
                    + rrow[pl.ds(ENT_DIM + k * L, L)] * tsw \
                    + rrow[pl.ds(2 * ENT_DIM + k * L, L)]
                acc = acc + jnp.abs(hv[k] * inv_h - out)
            return gamma - _lane_sum(acc, lane_iota)

        for c in range(min(NBUF, n_chunks)):
            start_gather(c, c)

        def chunk_body(c, carry):
            b = c % NBUF

            def wait_set(bs):
                return lambda: wait_gather(bs)

            lax.cond(b == 0, wait_set(0), wait_set(1))

            @plsc.parallel_loop(0, CHUNK // L, unroll=4)
            def _groups(g):
                def one_row(j, svec):
                    i = g * L + j
                    score = row_score(hbuf.at[b].at[i], tbuf.at[b].at[i],
                                      rbuf.at[b].at[i])
                    mask = lane_iota == lax.broadcast(j, (L,))
                    return jnp.where(mask, score, svec)
                svec = lax.fori_loop(
                    0, L, one_row, jnp.zeros((L,), jnp.float32))
                sbuf[pl.ds(c * CHUNK + g * L, L)] = svec

            nxt = c + NBUF

            def maybe_start(bs):
                def start():
                    lax.cond(nxt < n_chunks,
                             lambda: start_gather(nxt, bs), lambda: None)
                return start

            lax.cond(b == 0, maybe_start(0), maybe_start(1))
            return carry

        lax.fori_loop(0, n_chunks, chunk_body, jnp.int32(0))
        pltpu.sync_copy(sbuf, out_hbm.at[wid])

    return score_kernel(ent_table, rel_proc, h3, t3, r3)


def kernel(h, r, t, batch_type, ent_table, rel_table):
    batch = h.shape[0]
    b_per_w = batch // NW
    n_chunks = b_per_w // CHUNK
    rel_proc = _rel_prep(rel_table)
    h3 = h.reshape(NW, n_chunks, CHUNK)
    t3 = t.reshape(NW, n_chunks, CHUNK)
    r3 = r.reshape(NW, n_chunks, CHUNK)
    scores = _sc_score(ent_table, rel_proc, h3, t3, r3, batch)
    return scores.reshape(batch, 1)


# speedup vs baseline: 1.1087x; 1.1087x over previous
import functools

import jax
import jax.numpy as jnp
from jax import lax
from jax.experimental import pallas as pl
from jax.experimental.pallas import tpu as pltpu
from jax.experimental.pallas import tpu_sc as plsc

ENT_DIM = 128
REL_DIM = 384
EMBEDDING_RANGE = 0.109375
GAMMA = 12.0
PI = 3.141592653589793

NC = 2
NS = 16
NW = NC * NS
CHUNK = 64
NBUF = 2
L = 16


def _rel_prep_body(rel_ref, out_ref):
    rel = rel_ref[...]
    scale = rel[:, 0:ENT_DIM]
    translate = rel[:, ENT_DIM:2 * ENT_DIM]
    theta = rel[:, 2 * ENT_DIM:2 * ENT_DIM + ENT_DIM // 2]
    theta = theta * jnp.float32(PI / EMBEDDING_RANGE)
    r64 = lax.broadcasted_iota(jnp.int32, (ENT_DIM // 2, ENT_DIM), 0)
    c128 = lax.broadcasted_iota(jnp.int32, (ENT_DIM // 2, ENT_DIM), 1)
    p_dup = (c128 // 2 == r64).astype(jnp.float32)
    dot = functools.partial(
        jax.lax.dot_general,
        dimension_numbers=(((1,), (0,)), ((), ())),
        precision=jax.lax.Precision.HIGHEST,
    )
    cos_d = dot(jnp.cos(theta), p_dup)
    sin_d = dot(jnp.sin(theta), p_dup)
    lane = lax.broadcasted_iota(jnp.int32, (1, ENT_DIM), 1)
    sgn = jnp.where(lane % 2 == 0, jnp.float32(-1.0), jnp.float32(1.0))
    out_ref[...] = jnp.concatenate(
        [cos_d * scale, sgn * sin_d * scale, translate * scale], axis=1)


def _rel_prep(rel_table):
    n = rel_table.shape[0]
    return pl.pallas_call(
        _rel_prep_body,
        out_shape=jax.ShapeDtypeStruct((n, REL_DIM), jnp.float32),
    )(rel_table)


def _vec_rsqrt(s):
    one = jnp.full((L,), 1.0, jnp.float32)
    y = (one + one) / (one + s)
    half = jnp.full((L,), 0.5, jnp.float32) * s
    c15 = jnp.full((L,), 1.5, jnp.float32)
    for _ in range(7):
        y = y * (c15 - half * y * y)
    return y


def _lane_gather(vec, idx):
    return lax.gather(
        vec, idx[:, None],
        dimension_numbers=lax.GatherDimensionNumbers(
            offset_dims=(), collapsed_slice_dims=(0,), start_index_map=(0,)),
        slice_sizes=(1,),
        mode=lax.GatherScatterMode.PROMISE_IN_BOUNDS)


def _lane_sum(v, lane_iota):
    for sh in (8, 4, 2, 1):
        idx = lane_iota ^ jnp.full((L,), sh, jnp.int32)
        v = v + _lane_gather(v, idx)
    return v


def _sc_score(ent_table, rel_proc, h3, t3, r3, batch):
    b_per_w = batch // NW
    n_chunks = b_per_w // CHUNK
    mesh = plsc.VectorSubcoreMesh(core_axis_name="c", subcore_axis_name="s")

    @functools.partial(
        pl.kernel,
        mesh=mesh,
        out_type=jax.ShapeDtypeStruct((NW, b_per_w), jnp.float32),
        scratch_types=[
            pltpu.VMEM((n_chunks, CHUNK), jnp.int32),
            pltpu.VMEM((n_chunks, CHUNK), jnp.int32),
            pltpu.VMEM((n_chunks, CHUNK), jnp.int32),
            pltpu.VMEM((NBUF, CHUNK, ENT_DIM), jnp.float32),
            pltpu.VMEM((NBUF, CHUNK, ENT_DIM), jnp.float32),
            pltpu.VMEM((NBUF, CHUNK, REL_DIM), jnp.float32),
            pltpu.VMEM((b_per_w,), jnp.float32),
        ]
        + [pltpu.SemaphoreType.DMA] * (3 * NBUF),
    )
    def score_kernel(ent_hbm, rel_hbm, h_hbm, t_hbm, r_hbm, out_hbm,
                     hidx, tidx, ridx, hbuf, tbuf, rbuf, sbuf, *sems):
        wid = lax.axis_index("s") * NC + lax.axis_index("c")
        pltpu.sync_copy(h_hbm.at[wid], hidx)
        pltpu.sync_copy(t_hbm.at[wid], tidx)
        pltpu.sync_copy(r_hbm.at[wid], ridx)

        def start_gather(c, b):
            pltpu.make_async_copy(
                ent_hbm.at[hidx.at[c]], hbuf.at[b], sems[b]).start()
            pltpu.make_async_copy(
                ent_hbm.at[tidx.at[c]], tbuf.at[b], sems[NBUF + b]).start()
            pltpu.make_async_copy(
                rel_hbm.at[ridx.at[c]], rbuf.at[b], sems[2 * NBUF + b]).start()

        def wait_gather(b):
            pltpu.make_async_copy(
                ent_hbm.at[hidx.at[0]], hbuf.at[b], sems[b]).wait()
            pltpu.make_async_copy(
                ent_hbm.at[tidx.at[0]], tbuf.at[b], sems[NBUF + b]).wait()
            pltpu.make_async_copy(
                rel_hbm.at[ridx.at[0]], rbuf.at[b], sems[2 * NBUF + b]).wait()

        lane_iota = lax.iota(jnp.int32, L)
        one_i = jnp.full((L,), 1, jnp.int32)
        swap_idx = lane_iota ^ one_i
        gamma = jnp.full((L,), GAMMA, jnp.float32)

        def row_score(hrow, trow, rrow):
            hv, tv = [], []
            hh = jnp.zeros((L,), jnp.float32)
            tt = jnp.zeros((L,), jnp.float32)
            for k in range(ENT_DIM // L):
                hk = hrow[pl.ds(k * L, L)]
                tk = trow[pl.ds(k * L, L)]
                hv.append(hk)
                tv.append(tk)
                hh = hh + hk * hk
                tt = tt + tk * tk
            sh = _lane_sum(hh, lane_iota)
            st = _lane_sum(tt, lane_iota)
            inv_h = _vec_rsqrt(sh)
            inv_t = _vec_rsqrt(st)
            acc = jnp.zeros((L,), jnp.float32)
            for k in range(ENT_DIM // L):
                tn = tv[k] * inv_t
                tsw = _lane_gather(tn, swap_idx)
                out = rrow[pl.ds(k * L, L)] * tn \
                    + rrow[pl.ds(ENT_DIM + k * L, L)] * tsw \
                    + rrow[pl.ds(2 * ENT_DIM + k * L, L)]
                acc = acc + jnp.abs(hv[k] * inv_h - out)
            return gamma - _lane_sum(acc, lane_iota)

        for c in range(min(NBUF, n_chunks)):
            start_gather(c, c)
        for c in range(n_chunks):
            b = c % NBUF
            wait_gather(b)
            hb, tb, rb = hbuf.at[b], tbuf.at[b], rbuf.at[b]

            @plsc.parallel_loop(0, CHUNK // L, unroll=2)
            def _groups(g, hb=hb, tb=tb, rb=rb, c=c):
                def one_row(j, svec):
                    i = g * L + j
                    score = row_score(hb.at[i], tb.at[i], rb.at[i])
                    mask = lane_iota == lax.broadcast(j, (L,))
                    return jnp.where(mask, score, svec)
                svec = lax.fori_loop(
                    0, L, one_row, jnp.zeros((L,), jnp.float32))
                sbuf[pl.ds(c * CHUNK + g * L, L)] = svec

            nxt = c + NBUF
            if nxt < n_chunks:
                start_gather(nxt, b)
        pltpu.sync_copy(sbuf, out_hbm.at[wid])

    return score_kernel(ent_table, rel_proc, h3, t3, r3)


def kernel(h, r, t, batch_type, ent_table, rel_table):
    batch = h.shape[0]
    b_per_w = batch // NW
    n_chunks = b_per_w // CHUNK
    rel_proc = _rel_prep(rel_table)
    h3 = h.reshape(NW, n_chunks, CHUNK)
    t3 = t.reshape(NW, n_chunks, CHUNK)
    r3 = r.reshape(NW, n_chunks, CHUNK)
    scores = _sc_score(ent_table, rel_proc, h3, t3, r3, batch)
    return scores.reshape(batch, 1)


# speedup vs baseline: 1.1819x; 1.0661x over previous
import functools

import jax
import jax.numpy as jnp
from jax import lax
from jax.experimental import pallas as pl
from jax.experimental.pallas import tpu as pltpu
from jax.experimental.pallas import tpu_sc as plsc

ENT_DIM = 128
REL_DIM = 384
EMBEDDING_RANGE = 0.109375
GAMMA = 12.0
PI = 3.141592653589793

NC = 2
NS = 16
NW = NC * NS
CHUNK = 64
NBUF = 2
L = 16


def _rel_prep_body(rel_ref, out_ref):
    rel = rel_ref[...]
    scale = rel[:, 0:ENT_DIM]
    translate = rel[:, ENT_DIM:2 * ENT_DIM]
    theta = rel[:, 2 * ENT_DIM:2 * ENT_DIM + ENT_DIM // 2]
    theta = theta * jnp.float32(PI / EMBEDDING_RANGE)
    r64 = lax.broadcasted_iota(jnp.int32, (ENT_DIM // 2, ENT_DIM), 0)
    c128 = lax.broadcasted_iota(jnp.int32, (ENT_DIM // 2, ENT_DIM), 1)
    p_dup = (c128 // 2 == r64).astype(jnp.float32)
    dot = functools.partial(
        jax.lax.dot_general,
        dimension_numbers=(((1,), (0,)), ((), ())),
        precision=jax.lax.Precision.HIGHEST,
    )
    cos_d = dot(jnp.cos(theta), p_dup)
    sin_d = dot(jnp.sin(theta), p_dup)
    lane = lax.broadcasted_iota(jnp.int32, (1, ENT_DIM), 1)
    sgn = jnp.where(lane % 2 == 0, jnp.float32(-1.0), jnp.float32(1.0))
    out_ref[...] = jnp.concatenate(
        [cos_d * scale, sgn * sin_d * scale, translate * scale], axis=1)


def _rel_prep(rel_table):
    n = rel_table.shape[0]
    return pl.pallas_call(
        _rel_prep_body,
        out_shape=jax.ShapeDtypeStruct((n, REL_DIM), jnp.float32),
    )(rel_table)


def _vec_rsqrt(s):
    one = jnp.full((L,), 1.0, jnp.float32)
    y = (one + one) / (one + s)
    half = jnp.full((L,), 0.5, jnp.float32) * s
    c15 = jnp.full((L,), 1.5, jnp.float32)
    for _ in range(7):
        y = y * (c15 - half * y * y)
    return y


def _lane_gather(vec, idx):
    return lax.gather(
        vec, idx[:, None],
        dimension_numbers=lax.GatherDimensionNumbers(
            offset_dims=(), collapsed_slice_dims=(0,), start_index_map=(0,)),
        slice_sizes=(1,),
        mode=lax.GatherScatterMode.PROMISE_IN_BOUNDS)


def _lane_sum(v, lane_iota):
    for sh in (8, 4, 2, 1):
        idx = lane_iota ^ jnp.full((L,), sh, jnp.int32)
        v = v + _lane_gather(v, idx)
    return v


def _sc_score(ent_table, rel_proc, h3, t3, r3, batch):
    b_per_w = batch // NW
    n_chunks = b_per_w // CHUNK
    mesh = plsc.VectorSubcoreMesh(core_axis_name="c", subcore_axis_name="s")

    @functools.partial(
        pl.kernel,
        mesh=mesh,
        out_type=jax.ShapeDtypeStruct((NW, b_per_w), jnp.float32),
        scratch_types=[
            pltpu.VMEM((n_chunks, CHUNK), jnp.int32),
            pltpu.VMEM((n_chunks, CHUNK), jnp.int32),
            pltpu.VMEM((n_chunks, CHUNK), jnp.int32),
            pltpu.VMEM((NBUF, CHUNK, ENT_DIM), jnp.float32),
            pltpu.VMEM((NBUF, CHUNK, ENT_DIM), jnp.float32),
            pltpu.VMEM((NBUF, CHUNK, REL_DIM), jnp.float32),
            pltpu.VMEM((b_per_w,), jnp.float32),
        ]
        + [pltpu.SemaphoreType.DMA] * (3 * NBUF),
    )
    def score_kernel(ent_hbm, rel_hbm, h_hbm, t_hbm, r_hbm, out_hbm,
                     hidx, tidx, ridx, hbuf, tbuf, rbuf, sbuf, *sems):
        wid = lax.axis_index("s") * NC + lax.axis_index("c")
        pltpu.sync_copy(h_hbm.at[wid], hidx)
        pltpu.sync_copy(t_hbm.at[wid], tidx)
        pltpu.sync_copy(r_hbm.at[wid], ridx)

        def start_gather(c, b):
            pltpu.make_async_copy(
                ent_hbm.at[hidx.at[c]], hbuf.at[b], sems[b]).start()
            pltpu.make_async_copy(
                ent_hbm.at[tidx.at[c]], tbuf.at[b], sems[NBUF + b]).start()
            pltpu.make_async_copy(
                rel_hbm.at[ridx.at[c]], rbuf.at[b], sems[2 * NBUF + b]).start()

        def wait_gather(b):
            pltpu.make_async_copy(
                ent_hbm.at[hidx.at[0]], hbuf.at[b], sems[b]).wait()
            pltpu.make_async_copy(
                ent_hbm.at[tidx.at[0]], tbuf.at[b], sems[NBUF + b]).wait()
            pltpu.make_async_copy(
                rel_hbm.at[ridx.at[0]], rbuf.at[b], sems[2 * NBUF + b]).wait()

        lane_iota = lax.iota(jnp.int32, L)
        one_i = jnp.full((L,), 1, jnp.int32)
        swap_idx = lane_iota ^ one_i
        gamma = jnp.full((L,), GAMMA, jnp.float32)
        xor8 = lane_iota ^ jnp.full((L,), 8, jnp.int32)
        lo_mask = lane_iota < jnp.full((L,), 8, jnp.int32)
        idx_lo = lane_iota & jnp.full((L,), 7, jnp.int32)
        idx_hi = idx_lo + jnp.full((L,), 8, jnp.int32)

        def row_score(hrow, trow, rrow):
            hv, tv = [], []
            hh = jnp.zeros((L,), jnp.float32)
            tt = jnp.zeros((L,), jnp.float32)
            for k in range(ENT_DIM // L):
                hk = hrow[pl.ds(k * L, L)]
                tk = trow[pl.ds(k * L, L)]
                hv.append(hk)
                tv.append(tk)
                hh = hh + hk * hk
                tt = tt + tk * tk
            hh1 = hh + _lane_gather(hh, xor8)
            tt1 = tt + _lane_gather(tt, xor8)
            m = jnp.where(lo_mask, hh1, tt1)
            for sh_ in (4, 2, 1):
                m = m + _lane_gather(m, lane_iota ^ jnp.full((L,), sh_,
                                                             jnp.int32))
            y = _vec_rsqrt(m)
            inv_h = _lane_gather(y, idx_lo)
            inv_t = _lane_gather(y, idx_hi)
            acc = jnp.zeros((L,), jnp.float32)
            for k in range(ENT_DIM // L):
                tn = tv[k] * inv_t
                tsw = _lane_gather(tn, swap_idx)
                out = rrow[pl.ds(k * L, L)] * tn \
                    + rrow[pl.ds(ENT_DIM + k * L, L)] * tsw \
                    + rrow[pl.ds(2 * ENT_DIM + k * L, L)]
                acc = acc + jnp.abs(hv[k] * inv_h - out)
            return gamma - _lane_sum(acc, lane_iota)

        for c in range(min(NBUF, n_chunks)):
            start_gather(c, c)
        for c in range(n_chunks):
            b = c % NBUF
            wait_gather(b)
            hb, tb, rb = hbuf.at[b], tbuf.at[b], rbuf.at[b]

            @plsc.parallel_loop(0, CHUNK // L, unroll=2)
            def _groups(g, hb=hb, tb=tb, rb=rb, c=c):
                def one_row(j, svec):
                    i = g * L + j
                    score = row_score(hb.at[i], tb.at[i], rb.at[i])
                    mask = lane_iota == lax.broadcast(j, (L,))
                    return jnp.where(mask, score, svec)
                svec = lax.fori_loop(
                    0, L, one_row, jnp.zeros((L,), jnp.float32))
                sbuf[pl.ds(c * CHUNK + g * L, L)] = svec

            nxt = c + NBUF
            if nxt < n_chunks:
                start_gather(nxt, b)
        pltpu.sync_copy(sbuf, out_hbm.at[wid])

    return score_kernel(ent_table, rel_proc, h3, t3, r3)


def kernel(h, r, t, batch_type, ent_table, rel_table):
    batch = h.shape[0]
    b_per_w = batch // NW
    n_chunks = b_per_w // CHUNK
    rel_proc = _rel_prep(rel_table)
    h3 = h.reshape(NW, n_chunks, CHUNK)
    t3 = t.reshape(NW, n_chunks, CHUNK)
    r3 = r.reshape(NW, n_chunks, CHUNK)
    scores = _sc_score(ent_table, rel_proc, h3, t3, r3, batch)
    return scores.reshape(batch, 1)
